# baseline (speedup 1.0000x reference)
import functools

import jax
import jax.numpy as jnp
from jax import lax
from jax.experimental import pallas as pl
from jax.experimental.pallas import tpu as pltpu
from jax.experimental.pallas import tpu_sc as plsc

N_NODES = 10000
N_EDGES = 160000
D_FEAT = 128
TOTAL = N_NODES + 3 * N_EDGES

CHUNK = 96
NPG = 2
BLKROWS = NPG * CHUNK
RPW = 15360
LAST_START = TOTAL - RPW
NBLK = RPW // BLKROWS
NITER = NBLK // 2
TROWS = 632

_mesh = plsc.VectorSubcoreMesh(core_axis_name="c", subcore_axis_name="s")


@functools.partial(
    pl.kernel,
    mesh=_mesh,
    compiler_params=pltpu.CompilerParams(needs_layout_passes=False),
    out_type=[
        jax.ShapeDtypeStruct((TOTAL, D_FEAT), jnp.float32),
        jax.ShapeDtypeStruct((TOTAL,), jnp.int32),
    ],
    scratch_types=[
        pltpu.VMEM((2 * BLKROWS,), jnp.int32),
        pltpu.VMEM((2, BLKROWS, D_FEAT), jnp.float32),
        pltpu.VMEM((2 * BLKROWS,), jnp.int32),
        pltpu.VMEM_SHARED((N_NODES, D_FEAT), jnp.float32),
        pltpu.VMEM_SHARED((N_NODES,), jnp.int32),
        pltpu.SemaphoreType.DMA,
        pltpu.SemaphoreType.DMA,
        pltpu.SemaphoreType.DMA,
        pltpu.SemaphoreType.DMA,
        pltpu.SemaphoreType.DMA,
        pltpu.SemaphoreType.DMA,
    ],
)
def _gather_sc(x_hbm, idx_hbm, b_hbm, out_hbm, bout_hbm,
               idx_v, rows_v, vals_v, xs_sh, bt_sh,
               isem0, isem1, gsem0, gsem1, wsem0, wsem1):
    isems = (isem0, isem1)
    gsems = (gsem0, gsem1)
    wsems = (wsem0, wsem1)
    w = (lax.axis_index("s") * jnp.int32(2) + lax.axis_index("c")).astype(jnp.int32)
    wstart = jnp.minimum(w * jnp.int32(RPW), jnp.int32(LAST_START))

    sid = lax.axis_index("s").astype(jnp.int32)
    rstart = sid * jnp.int32(TROWS)

    def bounce_bt(start, size):
        pltpu.sync_copy(b_hbm.at[pl.ds(start, size)],
                        idx_v.at[pl.ds(jnp.int32(0), size)])
        pltpu.sync_copy(idx_v.at[pl.ds(jnp.int32(0), size)],
                        bt_sh.at[pl.ds(start, size)])

    @pl.when(sid < jnp.int32(15))
    def _():
        pltpu.sync_copy(x_hbm.at[pl.ds(rstart, TROWS)],
                        xs_sh.at[pl.ds(rstart, TROWS)])
        bounce_bt(rstart, 384)
        bounce_bt(rstart + jnp.int32(384), TROWS - 384)

    @pl.when(sid == jnp.int32(15))
    def _():
        last = jnp.int32(15 * TROWS)
        pltpu.sync_copy(x_hbm.at[pl.ds(last, N_NODES - 15 * TROWS)],
                        xs_sh.at[pl.ds(last, N_NODES - 15 * TROWS)])
        bounce_bt(last, 384)
        bounce_bt(last + jnp.int32(384), N_NODES - 15 * TROWS - 384)

    plsc.subcore_barrier()

    def parity_refs(p):
        return (rows_v.at[jnp.int32(p)],
                vals_v.at[pl.ds(jnp.int32(p * BLKROWS), BLKROWS)],
                idx_v.at[pl.ds(jnp.int32(p * BLKROWS), BLKROWS)])

    def drain_writes(p):
        rv, vv, _ = parity_refs(p)
        pltpu.make_async_copy(rv, out_hbm.at[pl.ds(jnp.int32(0), BLKROWS)],
                              wsems[p]).wait()
        pltpu.make_async_copy(vv, bout_hbm.at[pl.ds(jnp.int32(0), BLKROWS)],
                              wsems[p]).wait()

    def body(t, carry):
        for p in range(2):
            b = t * jnp.int32(2) + jnp.int32(p)
            rowbase = wstart + b * jnp.int32(BLKROWS)
            rv, vv, iv_blk = parity_refs(p)

            @pl.when(t > jnp.int32(0))
            def _(p=p):
                drain_writes(p)

            pltpu.make_async_copy(idx_hbm.at[pl.ds(jnp.int32(0), BLKROWS)],
                                  iv_blk, isems[p]).wait()
            for k in range(NPG):
                iv = idx_v.at[pl.ds(jnp.int32(p * BLKROWS + k * CHUNK), CHUNK)]
                pltpu.async_copy(
                    xs_sh.at[iv],
                    rows_v.at[jnp.int32(p), pl.ds(jnp.int32(k * CHUNK), CHUNK)],
                    gsems[p])
                pltpu.async_copy(
                    bt_sh.at[iv],
                    vals_v.at[pl.ds(jnp.int32(p * BLKROWS + k * CHUNK), CHUNK)],
                    gsems[p])
            for k in range(NPG):
                iv = idx_v.at[pl.ds(jnp.int32(p * BLKROWS + k * CHUNK), CHUNK)]
                pltpu.make_async_copy(
                    xs_sh.at[iv],
                    rows_v.at[jnp.int32(p), pl.ds(jnp.int32(k * CHUNK), CHUNK)],
                    gsems[p]).wait()
                pltpu.make_async_copy(
                    bt_sh.at[iv],
                    vals_v.at[pl.ds(jnp.int32(p * BLKROWS + k * CHUNK), CHUNK)],
                    gsems[p]).wait()

            @pl.when(t < jnp.int32(NITER - 1))
            def _(p=p, rowbase=rowbase, iv_blk=iv_blk):
                pltpu.async_copy(
                    idx_hbm.at[pl.ds(rowbase + jnp.int32(2 * BLKROWS), BLKROWS)],
                    iv_blk, isems[p])

            pltpu.async_copy(rv, out_hbm.at[pl.ds(rowbase, BLKROWS)], wsems[p])
            pltpu.async_copy(vv, bout_hbm.at[pl.ds(rowbase, BLKROWS)], wsems[p])
        return carry

    for p in range(2):
        _, _, iv_blk = parity_refs(p)
        pltpu.async_copy(
            idx_hbm.at[pl.ds(wstart + jnp.int32(p * BLKROWS), BLKROWS)],
            iv_blk, isems[p])
    lax.fori_loop(jnp.int32(0), jnp.int32(NITER), body, jnp.int32(0))
    for p in range(2):
        drain_writes(p)


def kernel(user_x, repost_edge_index, follow_edge_index, user_batch):
    idx_all = jnp.concatenate([
        jnp.arange(N_NODES, dtype=jnp.int32),
        repost_edge_index[0].astype(jnp.int32),
        repost_edge_index[1].astype(jnp.int32),
        follow_edge_index[1].astype(jnp.int32),
    ])
    batch_i32 = user_batch.astype(jnp.int32)
    out, bvals = _gather_sc(user_x, idx_all, batch_i32)
    e0 = N_NODES
    sender_batch = bvals[e0:e0 + N_EDGES].astype(user_batch.dtype)
    receiver_batch = bvals[e0 + N_EDGES:e0 + 2 * N_EDGES].astype(user_batch.dtype)
    follower_batch = bvals[e0 + 2 * N_EDGES:].astype(user_batch.dtype)
    return out, sender_batch, receiver_batch, follower_batch


# speedup vs baseline: 1.0011x; 1.0011x over previous
import functools

import jax
import jax.numpy as jnp
from jax import lax
from jax.experimental import pallas as pl
from jax.experimental.pallas import tpu as pltpu
from jax.experimental.pallas import tpu_sc as plsc

N_NODES = 10000
N_EDGES = 160000
D_FEAT = 128
TOTAL = N_NODES + 3 * N_EDGES

CHUNK = 48
NPG = 4
BLKROWS = NPG * CHUNK
RPW = 15360
LAST_START = TOTAL - RPW
NBLK = RPW // BLKROWS
NITER = NBLK // 2
TROWS = 632

_mesh = plsc.VectorSubcoreMesh(core_axis_name="c", subcore_axis_name="s")


@functools.partial(
    pl.kernel,
    mesh=_mesh,
    compiler_params=pltpu.CompilerParams(needs_layout_passes=False),
    out_type=[
        jax.ShapeDtypeStruct((TOTAL, D_FEAT), jnp.float32),
        jax.ShapeDtypeStruct((TOTAL,), jnp.int32),
    ],
    scratch_types=[
        pltpu.VMEM((2 * BLKROWS,), jnp.int32),
        pltpu.VMEM((2, BLKROWS, D_FEAT), jnp.float32),
        pltpu.VMEM((2 * BLKROWS,), jnp.int32),
        pltpu.VMEM_SHARED((N_NODES, D_FEAT), jnp.float32),
        pltpu.VMEM_SHARED((N_NODES,), jnp.int32),
        pltpu.SemaphoreType.DMA,
        pltpu.SemaphoreType.DMA,
        pltpu.SemaphoreType.DMA,
        pltpu.SemaphoreType.DMA,
        pltpu.SemaphoreType.DMA,
        pltpu.SemaphoreType.DMA,
    ],
)
def _gather_sc(x_hbm, idx_hbm, b_hbm, out_hbm, bout_hbm,
               idx_v, rows_v, vals_v, xs_sh, bt_sh,
               isem0, isem1, gsem0, gsem1, wsem0, wsem1):
    isems = (isem0, isem1)
    gsems = (gsem0, gsem1)
    wsems = (wsem0, wsem1)
    w = (lax.axis_index("s") * jnp.int32(2) + lax.axis_index("c")).astype(jnp.int32)
    wstart = jnp.minimum(w * jnp.int32(RPW), jnp.int32(LAST_START))

    sid = lax.axis_index("s").astype(jnp.int32)
    rstart = sid * jnp.int32(TROWS)

    def bounce_bt(start, size):
        pltpu.sync_copy(b_hbm.at[pl.ds(start, size)],
                        idx_v.at[pl.ds(jnp.int32(0), size)])
        pltpu.sync_copy(idx_v.at[pl.ds(jnp.int32(0), size)],
                        bt_sh.at[pl.ds(start, size)])

    @pl.when(sid < jnp.int32(15))
    def _():
        pltpu.sync_copy(x_hbm.at[pl.ds(rstart, TROWS)],
                        xs_sh.at[pl.ds(rstart, TROWS)])
        bounce_bt(rstart, 384)
        bounce_bt(rstart + jnp.int32(384), TROWS - 384)

    @pl.when(sid == jnp.int32(15))
    def _():
        last = jnp.int32(15 * TROWS)
        pltpu.sync_copy(x_hbm.at[pl.ds(last, N_NODES - 15 * TROWS)],
                        xs_sh.at[pl.ds(last, N_NODES - 15 * TROWS)])
        bounce_bt(last, 384)
        bounce_bt(last + jnp.int32(384), N_NODES - 15 * TROWS - 384)

    plsc.subcore_barrier()

    def parity_refs(p):
        return (rows_v.at[jnp.int32(p)],
                vals_v.at[pl.ds(jnp.int32(p * BLKROWS), BLKROWS)],
                idx_v.at[pl.ds(jnp.int32(p * BLKROWS), BLKROWS)])

    def drain_writes(p):
        rv, vv, _ = parity_refs(p)
        pltpu.make_async_copy(rv, out_hbm.at[pl.ds(jnp.int32(0), BLKROWS)],
                              wsems[p]).wait()
        pltpu.make_async_copy(vv, bout_hbm.at[pl.ds(jnp.int32(0), BLKROWS)],
                              wsems[p]).wait()

    def body(t, carry):
        for p in range(2):
            b = t * jnp.int32(2) + jnp.int32(p)
            rowbase = wstart + b * jnp.int32(BLKROWS)
            rv, vv, iv_blk = parity_refs(p)

            @pl.when(t > jnp.int32(0))
            def _(p=p):
                drain_writes(p)

            pltpu.make_async_copy(idx_hbm.at[pl.ds(jnp.int32(0), BLKROWS)],
                                  iv_blk, isems[p]).wait()
            for k in range(NPG):
                iv = idx_v.at[pl.ds(jnp.int32(p * BLKROWS + k * CHUNK), CHUNK)]
                pltpu.async_copy(
                    xs_sh.at[iv],
                    rows_v.at[jnp.int32(p), pl.ds(jnp.int32(k * CHUNK), CHUNK)],
                    gsems[p])
                pltpu.async_copy(
                    bt_sh.at[iv],
                    vals_v.at[pl.ds(jnp.int32(p * BLKROWS + k * CHUNK), CHUNK)],
                    gsems[p])
            for k in range(NPG):
                iv = idx_v.at[pl.ds(jnp.int32(p * BLKROWS + k * CHUNK), CHUNK)]
                pltpu.make_async_copy(
                    xs_sh.at[iv],
                    rows_v.at[jnp.int32(p), pl.ds(jnp.int32(k * CHUNK), CHUNK)],
                    gsems[p]).wait()
                pltpu.make_async_copy(
                    bt_sh.at[iv],
                    vals_v.at[pl.ds(jnp.int32(p * BLKROWS + k * CHUNK), CHUNK)],
                    gsems[p]).wait()

            @pl.when(t < jnp.int32(NITER - 1))
            def _(p=p, rowbase=rowbase, iv_blk=iv_blk):
                pltpu.async_copy(
                    idx_hbm.at[pl.ds(rowbase + jnp.int32(2 * BLKROWS), BLKROWS)],
                    iv_blk, isems[p])

            pltpu.async_copy(rv, out_hbm.at[pl.ds(rowbase, BLKROWS)], wsems[p])
            pltpu.async_copy(vv, bout_hbm.at[pl.ds(rowbase, BLKROWS)], wsems[p])
        return carry

    for p in range(2):
        _, _, iv_blk = parity_refs(p)
        pltpu.async_copy(
            idx_hbm.at[pl.ds(wstart + jnp.int32(p * BLKROWS), BLKROWS)],
            iv_blk, isems[p])
    lax.fori_loop(jnp.int32(0), jnp.int32(NITER), body, jnp.int32(0))
    for p in range(2):
        drain_writes(p)


def kernel(user_x, repost_edge_index, follow_edge_index, user_batch):
    idx_all = jnp.concatenate([
        jnp.arange(N_NODES, dtype=jnp.int32),
        repost_edge_index[0].astype(jnp.int32),
        repost_edge_index[1].astype(jnp.int32),
        follow_edge_index[1].astype(jnp.int32),
    ])
    batch_i32 = user_batch.astype(jnp.int32)
    out, bvals = _gather_sc(user_x, idx_all, batch_i32)
    e0 = N_NODES
    sender_batch = bvals[e0:e0 + N_EDGES].astype(user_batch.dtype)
    receiver_batch = bvals[e0 + N_EDGES:e0 + 2 * N_EDGES].astype(user_batch.dtype)
    follower_batch = bvals[e0 + 2 * N_EDGES:].astype(user_batch.dtype)
    return out, sender_batch, receiver_batch, follower_batch
